# baseline (speedup 1.0000x reference)
import functools

import jax
import jax.numpy as jnp
from jax import lax
from jax.experimental import pallas as pl
from jax.experimental.pallas import tpu as pltpu
from jax.experimental.pallas import tpu_sc as plsc

NC = 2
NS = 16
NW = NC * NS

BATCH = 16384
EMB = 64
HALF = EMB // 2
PACK = 128
B_PER_W = BATCH // NW

PCB = 16384


def _pack_body(b0, b1, b2, b3, out_ref):
    half = jnp.uint32(0x8000)
    himask = jnp.uint32(0xFFFF0000)
    quarters = []
    for blk in (b0, b1, b2, b3):
        ulo = lax.bitcast_convert_type(blk[0:HALF, :], jnp.uint32)
        uhi = lax.bitcast_convert_type(blk[HALF:EMB, :], jnp.uint32)
        quarters.append(((ulo + half) >> 16) | ((uhi + half) & himask))
    pk = jnp.concatenate(quarters, axis=0)
    out_ref[...] = lax.bitcast_convert_type(pk, jnp.float32).T


def _pack(tabT):
    n = tabT.shape[1]
    nblk = pl.cdiv(pl.cdiv(n, 4), PCB)
    m = nblk * PCB
    last = pl.cdiv(n, PCB) - 1

    def mk_map(k):
        return lambda i: (0, jnp.minimum(i + k * nblk, last))

    return pl.pallas_call(
        _pack_body,
        grid=(nblk,),
        in_specs=[pl.BlockSpec((EMB, PCB), mk_map(k)) for k in range(4)],
        out_specs=pl.BlockSpec((PCB, PACK), lambda i: (i, 0)),
        out_shape=jax.ShapeDtypeStruct((m, PACK), jnp.float32),
    )(tabT, tabT, tabT, tabT), m


def _sc_gather(tab2, idx):
    mesh = plsc.VectorSubcoreMesh(core_axis_name="c", subcore_axis_name="s")

    @functools.partial(
        pl.kernel,
        mesh=mesh,
        out_type=jax.ShapeDtypeStruct((BATCH, PACK), jnp.float32),
        scratch_types=[
            pltpu.VMEM((B_PER_W,), jnp.int32),
            pltpu.VMEM((B_PER_W, PACK), jnp.float32),
            pltpu.SemaphoreType.DMA,
        ],
    )
    def gather_kernel(tab_hbm, idx_hbm, emb_hbm, idx_v, rows_v, sem):
        wid = lax.axis_index("s") * NC + lax.axis_index("c")
        base = wid * B_PER_W
        pltpu.sync_copy(idx_hbm.at[pl.ds(base, B_PER_W)], idx_v)
        pltpu.async_copy(tab_hbm.at[idx_v], rows_v, sem).wait()
        pltpu.sync_copy(rows_v, emb_hbm.at[pl.ds(base, B_PER_W)])

    return gather_kernel(tab2, idx)


BM = 2048


def _quarter(g, q):
    s0 = g[:, 0:32]
    s1 = g[:, 32:64]
    s2 = g[:, 64:96]
    s3 = g[:, 96:128]
    x01 = jnp.where(q == 1, s1, s0)
    x23 = jnp.where(q == 3, s3, s2)
    return jnp.where(q >= 2, x23, x01)


def _unpack(x):
    u = lax.bitcast_convert_type(x, jnp.uint32)
    lo = lax.bitcast_convert_type(u << 16, jnp.float32)
    hi = lax.bitcast_convert_type(u & jnp.uint32(0xFFFF0000), jnp.float32)
    return lo, hi


def _mlp_body(gu_ref, gi_ref, uq_ref, iq_ref, w1_ref, b1_ref, w2_ref, b2_ref,
              w3_ref, b3_ref, o_ref):
    ulo, uhi = _unpack(_quarter(gu_ref[...], uq_ref[...]))
    ilo, ihi = _unpack(_quarter(gi_ref[...], iq_ref[...]))
    x = jnp.concatenate([ulo, uhi, ilo, ihi], axis=1)
    h = jnp.dot(x, w1_ref[...], preferred_element_type=jnp.float32)
    h = jnp.maximum(h + b1_ref[...], 0.0)
    h2 = jnp.dot(h, w2_ref[...], preferred_element_type=jnp.float32)
    h2 = jnp.maximum(h2 + b2_ref[...], 0.0)
    out = jnp.dot(h2, w3_ref[...], preferred_element_type=jnp.float32)
    o_ref[...] = out + b3_ref[...]


def _tc_mlp(gu, gi, uq, iq, W1, b1, W2, b2, W3, b3):
    h1 = W1.shape[1]
    h2 = W2.shape[1]
    grid = (BATCH // BM,)
    out = pl.pallas_call(
        _mlp_body,
        grid=grid,
        in_specs=[
            pl.BlockSpec((BM, PACK), lambda i: (i, 0)),
            pl.BlockSpec((BM, PACK), lambda i: (i, 0)),
            pl.BlockSpec((BM, HALF), lambda i: (i, 0)),
            pl.BlockSpec((BM, HALF), lambda i: (i, 0)),
            pl.BlockSpec((2 * EMB, h1), lambda i: (0, 0)),
            pl.BlockSpec((1, h1), lambda i: (0, 0)),
            pl.BlockSpec((h1, h2), lambda i: (0, 0)),
            pl.BlockSpec((1, h2), lambda i: (0, 0)),
            pl.BlockSpec((h2, 1), lambda i: (0, 0)),
            pl.BlockSpec((1, 1), lambda i: (0, 0)),
        ],
        out_specs=pl.BlockSpec((BM, 1), lambda i: (i, 0)),
        out_shape=jax.ShapeDtypeStruct((BATCH, 1), jnp.float32),
    )(gu, gi, uq, iq, W1, b1.reshape(1, h1), W2, b2.reshape(1, h2), W3,
      b3.reshape(1, 1))
    return out.reshape(BATCH)


def kernel(user_ids, item_ids, user_table, item_table, W1, b1, W2, b2, W3, b3):
    uid = user_ids.astype(jnp.int32)
    iid = item_ids.astype(jnp.int32)
    itab2, mi = _pack(item_table.T)
    iq = iid // mi
    gi = _sc_gather(itab2, iid - iq * mi)
    utab2, mu = _pack(user_table.T)
    uq = uid // mu
    gu = _sc_gather(utab2, uid - uq * mu)
    uqb = jnp.broadcast_to(uq.reshape(BATCH, 1), (BATCH, HALF))
    iqb = jnp.broadcast_to(iq.reshape(BATCH, 1), (BATCH, HALF))
    return _tc_mlp(gu, gi, uqb, iqb, W1, b1, W2, b2, W3, b3)


# speedup vs baseline: 1.0645x; 1.0645x over previous
import functools

import jax
import jax.numpy as jnp
from jax import lax
from jax.experimental import pallas as pl
from jax.experimental.pallas import tpu as pltpu
from jax.experimental.pallas import tpu_sc as plsc

NC = 2
NS = 16
NW = NC * NS

BATCH = 16384
EMB = 64
HALF = EMB // 2
PACK = 128
B_PER_W = BATCH // NW

PCB = 16384


def _pack_body(b0, b1, b2, b3, out_ref):
    half = jnp.uint32(0x8000)
    himask = jnp.uint32(0xFFFF0000)
    quarters = []
    for blk in (b0, b1, b2, b3):
        ulo = lax.bitcast_convert_type(blk[0:HALF, :], jnp.uint32)
        uhi = lax.bitcast_convert_type(blk[HALF:EMB, :], jnp.uint32)
        quarters.append(((ulo + half) >> 16) | ((uhi + half) & himask))
    pk = jnp.concatenate(quarters, axis=0)
    out_ref[...] = lax.bitcast_convert_type(pk, jnp.float32).T


def _pack(tabT):
    n = tabT.shape[1]
    nblk = pl.cdiv(pl.cdiv(n, 4), PCB)
    m = nblk * PCB
    last = pl.cdiv(n, PCB) - 1

    def mk_map(k):
        return lambda i: (0, jnp.minimum(i + k * nblk, last))

    return pl.pallas_call(
        _pack_body,
        grid=(nblk,),
        in_specs=[pl.BlockSpec((EMB, PCB), mk_map(k)) for k in range(4)],
        out_specs=pl.BlockSpec((PCB, PACK), lambda i: (i, 0)),
        out_shape=jax.ShapeDtypeStruct((m, PACK), jnp.float32),
    )(tabT, tabT, tabT, tabT), m


def _sc_gather(tab2, idx):
    mesh = plsc.VectorSubcoreMesh(core_axis_name="c", subcore_axis_name="s")

    @functools.partial(
        pl.kernel,
        mesh=mesh,
        out_type=jax.ShapeDtypeStruct((BATCH, PACK), jnp.float32),
        scratch_types=[
            pltpu.VMEM((B_PER_W,), jnp.int32),
            pltpu.VMEM((B_PER_W, PACK), jnp.float32),
            pltpu.SemaphoreType.DMA,
        ],
    )
    def gather_kernel(tab_hbm, idx_hbm, emb_hbm, idx_v, rows_v, sem):
        wid = lax.axis_index("s") * NC + lax.axis_index("c")
        base = wid * B_PER_W
        pltpu.sync_copy(idx_hbm.at[pl.ds(base, B_PER_W)], idx_v)
        pltpu.async_copy(tab_hbm.at[idx_v], rows_v, sem).wait()
        pltpu.sync_copy(rows_v, emb_hbm.at[pl.ds(base, B_PER_W)])

    return gather_kernel(tab2, idx)


BM = 2048


def _quarter(g, q):
    s0 = g[:, 0:32]
    s1 = g[:, 32:64]
    s2 = g[:, 64:96]
    s3 = g[:, 96:128]
    x01 = jnp.where(q == 1, s1, s0)
    x23 = jnp.where(q == 3, s3, s2)
    return jnp.where(q >= 2, x23, x01)


def _unpack(x):
    u = lax.bitcast_convert_type(x, jnp.uint32)
    lo = lax.bitcast_convert_type(u << 16, jnp.float32)
    hi = lax.bitcast_convert_type(u & jnp.uint32(0xFFFF0000), jnp.float32)
    return lo, hi


def _mlp_body(gu_ref, gi_ref, uq_ref, iq_ref, w1_ref, b1_ref, w2_ref, b2_ref,
              w3_ref, b3_ref, o_ref):
    uqb = jnp.broadcast_to(uq_ref[...], (BM, HALF))
    iqb = jnp.broadcast_to(iq_ref[...], (BM, HALF))
    ulo, uhi = _unpack(_quarter(gu_ref[...], uqb))
    ilo, ihi = _unpack(_quarter(gi_ref[...], iqb))
    x = jnp.concatenate([ulo, uhi, ilo, ihi], axis=1)
    h = jnp.dot(x, w1_ref[...], preferred_element_type=jnp.float32)
    h = jnp.maximum(h + b1_ref[...], 0.0)
    h2 = jnp.dot(h, w2_ref[...], preferred_element_type=jnp.float32)
    h2 = jnp.maximum(h2 + b2_ref[...], 0.0)
    out = jnp.dot(h2, w3_ref[...], preferred_element_type=jnp.float32)
    o_ref[...] = out + b3_ref[...]


def _tc_mlp(gu, gi, uq, iq, W1, b1, W2, b2, W3, b3):
    h1 = W1.shape[1]
    h2 = W2.shape[1]
    grid = (BATCH // BM,)
    out = pl.pallas_call(
        _mlp_body,
        grid=grid,
        in_specs=[
            pl.BlockSpec((BM, PACK), lambda i: (i, 0)),
            pl.BlockSpec((BM, PACK), lambda i: (i, 0)),
            pl.BlockSpec((BM, 1), lambda i: (i, 0)),
            pl.BlockSpec((BM, 1), lambda i: (i, 0)),
            pl.BlockSpec((2 * EMB, h1), lambda i: (0, 0)),
            pl.BlockSpec((1, h1), lambda i: (0, 0)),
            pl.BlockSpec((h1, h2), lambda i: (0, 0)),
            pl.BlockSpec((1, h2), lambda i: (0, 0)),
            pl.BlockSpec((h2, 1), lambda i: (0, 0)),
            pl.BlockSpec((1, 1), lambda i: (0, 0)),
        ],
        out_specs=pl.BlockSpec((BM, 1), lambda i: (i, 0)),
        out_shape=jax.ShapeDtypeStruct((BATCH, 1), jnp.float32),
    )(gu, gi, uq, iq, W1, b1.reshape(1, h1), W2, b2.reshape(1, h2), W3,
      b3.reshape(1, 1))
    return out.reshape(BATCH)


def kernel(user_ids, item_ids, user_table, item_table, W1, b1, W2, b2, W3, b3):
    uid = user_ids.astype(jnp.int32)
    iid = item_ids.astype(jnp.int32)
    itab2, mi = _pack(item_table.T)
    iq = iid // mi
    gi = _sc_gather(itab2, iid - iq * mi)
    user_table, _ = lax.optimization_barrier((user_table, itab2))
    utab2, mu = _pack(user_table.T)
    uq = uid // mu
    gu = _sc_gather(utab2, uid - uq * mu)
    return _tc_mlp(gu, gi, uq.reshape(BATCH, 1), iq.reshape(BATCH, 1),
                   W1, b1, W2, b2, W3, b3)
